# Initial kernel scaffold; baseline (speedup 1.0000x reference)
import jax
import jax.numpy as jnp
from jax.experimental import pallas as pl


def kernel(SInterBatch, TInterBatch, SUserTable, TUserTable, SItemTable, TItemTable, W_dan, b_dan, W_srec, b_srec, W_trec, b_trec, W_enc, b_enc, W_dec, b_dec):
    raise NotImplementedError("write your pallas kernel here")


# speedup vs baseline: 1.2636x; 1.2636x over previous
import functools

import jax
import jax.numpy as jnp
from jax import lax
from jax.experimental import pallas as pl
from jax.experimental.pallas import tpu as pltpu
from jax.experimental.pallas import tpu_sc as plsc

B = 16384
D = 64
NW = 32
ROWS_PER_W = B // NW
CHUNK = 128
CHUNKS_PER_W = ROWS_PER_W // CHUNK
IDX_ROWS = B // CHUNK


def _sc_gather_body(su_t, tu_t, si_t, ti_t,
                    i_su, i_tu, i_sp, i_sn, i_tp, i_tn,
                    o_su, o_tu, o_sp, o_sn, o_tp, o_tn,
                    idx_v, rows_v, sem):
  wid = lax.axis_index("s") * 2 + lax.axis_index("c")
  base = wid * CHUNKS_PER_W
  jobs = ((su_t, i_su, o_su), (tu_t, i_tu, o_tu),
          (si_t, i_sp, o_sp), (si_t, i_sn, o_sn),
          (ti_t, i_tp, o_tp), (ti_t, i_tn, o_tn))
  for tab, idx, out in jobs:
    pltpu.sync_copy(idx.at[pl.ds(base, CHUNKS_PER_W), :], idx_v)
    copies = [
        pltpu.async_copy(tab.at[idx_v.at[j]], rows_v.at[j], sem)
        for j in range(CHUNKS_PER_W)
    ]
    for c in copies:
      c.wait()
    pltpu.sync_copy(rows_v, out.at[pl.ds(base, CHUNKS_PER_W), :, :])


def _sc_gather(su_t, tu_t, si_t, ti_t, i_su, i_tu, i_sp, i_sn, i_tp, i_tn):
  out3 = jax.ShapeDtypeStruct((IDX_ROWS, CHUNK, D), jnp.float32)
  mesh = plsc.VectorSubcoreMesh(core_axis_name="c", subcore_axis_name="s")
  k = functools.partial(
      pl.kernel,
      mesh=mesh,
      compiler_params=pltpu.CompilerParams(use_tc_tiling_on_sc=False),
      out_type=[out3] * 6,
      scratch_types=[
          pltpu.VMEM((CHUNKS_PER_W, CHUNK), jnp.int32),
          pltpu.VMEM((CHUNKS_PER_W, CHUNK, D), jnp.float32),
          pltpu.SemaphoreType.DMA,
      ],
  )(_sc_gather_body)
  return k(su_t, tu_t, si_t, ti_t, i_su, i_tu, i_sp, i_sn, i_tp, i_tn)


BLK = 2048


def _tc_body(su, tu, sp, sn, tp, tn,
             wdan, bdan, wenc, benc, wdec, bdec,
             ws1, ws2, wt1, wt2, bs, bt,
             o_spos, o_tpos, o_sneg, o_tneg,
             o_aspos, o_asneg, o_atpos, o_atneg):
  dot = functools.partial(lax.dot, precision=lax.Precision.HIGHEST,
                          preferred_element_type=jnp.float32)
  sdi = jnp.maximum(dot(su[...], wdan[...]) + bdan[...], 0.0)
  tdi = jnp.maximum(dot(tu[...], wdan[...]) + bdan[...], 0.0)
  scd = dot(jnp.maximum(dot(sdi, wenc[...]) + benc[...], 0.0), wdec[...]) + bdec[...]
  tcd = dot(jnp.maximum(dot(tdi, wenc[...]) + benc[...], 0.0), wdec[...]) + bdec[...]
  a_s = dot(sdi, ws1[...])
  c_s = dot(scd, ws1[...])
  p_s = dot(sp[...], ws2[...])
  n_s = dot(sn[...], ws2[...])
  a_t = dot(tdi, wt1[...])
  c_t = dot(tcd, wt1[...])
  p_t = dot(tp[...], wt2[...])
  n_t = dot(tn[...], wt2[...])
  bsv = bs[0, 0]
  btv = bt[0, 0]
  o_spos[...] = a_s + p_s + bsv
  o_tpos[...] = a_t + p_t + btv
  o_sneg[...] = a_s + n_s + bsv
  o_tneg[...] = a_t + n_t + btv
  o_aspos[...] = c_s + p_s + bsv
  o_asneg[...] = c_s + n_s + bsv
  o_atpos[...] = c_t + p_t + btv
  o_atneg[...] = c_t + n_t + btv


def _tc_heads(su_g, tu_g, sp_g, sn_g, tp_g, tn_g,
              wdan, bdan, wenc, benc, wdec, bdec,
              ws1, ws2, wt1, wt2, bs, bt):
  row_spec = pl.BlockSpec((BLK, D), lambda i: (i, 0))
  full = lambda a: pl.BlockSpec(a.shape, lambda i: (0,) * a.ndim)
  out_spec = pl.BlockSpec((BLK, 1), lambda i: (i, 0))
  return pl.pallas_call(
      _tc_body,
      grid=(B // BLK,),
      in_specs=[row_spec] * 6 + [full(w) for w in
                                 (wdan, bdan, wenc, benc, wdec, bdec,
                                  ws1, ws2, wt1, wt2, bs, bt)],
      out_specs=[out_spec] * 8,
      out_shape=[jax.ShapeDtypeStruct((B, 1), jnp.float32)] * 8,
  )(su_g, tu_g, sp_g, sn_g, tp_g, tn_g,
    wdan, bdan, wenc, benc, wdec, bdec, ws1, ws2, wt1, wt2, bs, bt)


def kernel(SInterBatch, TInterBatch, SUserTable, TUserTable, SItemTable,
           TItemTable, W_dan, b_dan, W_srec, b_srec, W_trec, b_trec,
           W_enc, b_enc, W_dec, b_dec):
  idx = lambda a: a.astype(jnp.int32).reshape(IDX_ROWS, CHUNK)
  gathered = _sc_gather(
      SUserTable, TUserTable, SItemTable, TItemTable,
      idx(SInterBatch[0]), idx(TInterBatch[0]),
      idx(SInterBatch[1]), idx(SInterBatch[2]),
      idx(TInterBatch[1]), idx(TInterBatch[2]))
  su_g, tu_g, sp_g, sn_g, tp_g, tn_g = (g.reshape(B, D) for g in gathered)

  outs = _tc_heads(
      su_g, tu_g, sp_g, sn_g, tp_g, tn_g,
      W_dan, b_dan.reshape(1, -1), W_enc, b_enc.reshape(1, -1),
      W_dec, b_dec.reshape(1, -1),
      W_srec[:32], W_srec[32:], W_trec[:32], W_trec[32:],
      b_srec.reshape(1, 1), b_trec.reshape(1, 1))
  (spos, tpos, sneg, tneg, aspos, asneg, atpos, atneg) = outs
  return jnp.concatenate(
      [spos, tpos, sneg, tneg, aspos, asneg, atpos, atneg], axis=0)


# speedup vs baseline: 1.7865x; 1.4138x over previous
import functools

import jax
import jax.numpy as jnp
from jax import lax
from jax.experimental import pallas as pl
from jax.experimental.pallas import tpu as pltpu
from jax.experimental.pallas import tpu_sc as plsc

B = 16384
D = 64
NW = 32
ROWS_PER_W = B // NW
CHUNK = 128
CHUNKS_PER_W = ROWS_PER_W // CHUNK
IDX_ROWS = B // CHUNK


def _sc_gather_body(su_t, tu_t, si_t, ti_t,
                    i_su, i_tu, i_sp, i_sn, i_tp, i_tn,
                    o_u, o_s, o_t,
                    idx_v, rows_v, sem):
  wid = lax.axis_index("s") * 2 + lax.axis_index("c")
  base = wid * CHUNKS_PER_W
  row0 = wid * ROWS_PER_W
  jobs = ((su_t, i_su, o_u, 0), (tu_t, i_tu, o_u, D),
          (si_t, i_sp, o_s, 0), (si_t, i_sn, o_s, D),
          (ti_t, i_tp, o_t, 0), (ti_t, i_tn, o_t, D))
  for tab, idx, out, col in jobs:
    pltpu.sync_copy(idx.at[pl.ds(base, CHUNKS_PER_W), :], idx_v)
    copies = [
        pltpu.async_copy(tab.at[idx_v.at[j]],
                         rows_v.at[pl.ds(j * CHUNK, CHUNK), :], sem)
        for j in range(CHUNKS_PER_W)
    ]
    for c in copies:
      c.wait()
    pltpu.sync_copy(rows_v, out.at[pl.ds(row0, ROWS_PER_W), pl.ds(col, D)])


def _sc_gather(su_t, tu_t, si_t, ti_t, i_su, i_tu, i_sp, i_sn, i_tp, i_tn):
  out2 = jax.ShapeDtypeStruct((B, 2 * D), jnp.float32)
  mesh = plsc.VectorSubcoreMesh(core_axis_name="c", subcore_axis_name="s")
  k = functools.partial(
      pl.kernel,
      mesh=mesh,
      compiler_params=pltpu.CompilerParams(use_tc_tiling_on_sc=False),
      out_type=[out2] * 3,
      scratch_types=[
          pltpu.VMEM((CHUNKS_PER_W, CHUNK), jnp.int32),
          pltpu.VMEM((ROWS_PER_W, D), jnp.float32),
          pltpu.SemaphoreType.DMA,
      ],
  )(_sc_gather_body)
  return k(su_t, tu_t, si_t, ti_t, i_su, i_tu, i_sp, i_sn, i_tp, i_tn)


BLK = 4096


def _tc_body(u, p, q,
             wdanT, bdanC, wencT, bencC, wdecT, bdecC,
             ws1T, ws2T, wt1T, wt2T, bs, bt, out):
  dn = (((1,), (0,)), ((), ()))
  dt = (((1,), (1,)), ((), ()))
  dot = functools.partial(lax.dot_general,
                          precision=lax.Precision.HIGHEST,
                          preferred_element_type=jnp.float32)
  su = u[:, :D]
  tu = u[:, D:]
  sp = p[:, :D]
  sn = p[:, D:]
  tp = q[:, :D]
  tn = q[:, D:]
  sdiT = jnp.maximum(dot(wdanT[...], su, dt) + bdanC[...], 0.0)
  tdiT = jnp.maximum(dot(wdanT[...], tu, dt) + bdanC[...], 0.0)
  scdT = dot(wdecT[...],
             jnp.maximum(dot(wencT[...], sdiT, dn) + bencC[...], 0.0),
             dn) + bdecC[...]
  tcdT = dot(wdecT[...],
             jnp.maximum(dot(wencT[...], tdiT, dn) + bencC[...], 0.0),
             dn) + bdecC[...]
  a_s = dot(ws1T[...], sdiT, dn)
  c_s = dot(ws1T[...], scdT, dn)
  p_s = dot(ws2T[...], sp, dt)
  n_s = dot(ws2T[...], sn, dt)
  a_t = dot(wt1T[...], tdiT, dn)
  c_t = dot(wt1T[...], tcdT, dn)
  p_t = dot(wt2T[...], tp, dt)
  n_t = dot(wt2T[...], tn, dt)
  bsv = bs[0, 0]
  btv = bt[0, 0]
  out[...] = jnp.concatenate(
      [a_s + p_s + bsv, a_t + p_t + btv,
       a_s + n_s + bsv, a_t + n_t + btv,
       c_s + p_s + bsv, c_s + n_s + bsv,
       c_t + p_t + btv, c_t + n_t + btv], axis=0)


def _tc_heads(u, p, q, wdanT, bdanC, wencT, bencC, wdecT, bdecC,
              ws1T, ws2T, wt1T, wt2T, bs, bt):
  row_spec = pl.BlockSpec((BLK, 2 * D), lambda i: (i, 0))
  full = lambda a: pl.BlockSpec(a.shape, lambda i: (0,) * a.ndim)
  return pl.pallas_call(
      _tc_body,
      grid=(B // BLK,),
      in_specs=[row_spec] * 3 + [full(w) for w in
                                 (wdanT, bdanC, wencT, bencC, wdecT, bdecC,
                                  ws1T, ws2T, wt1T, wt2T, bs, bt)],
      out_specs=pl.BlockSpec((8, BLK), lambda i: (0, i)),
      out_shape=jax.ShapeDtypeStruct((8, B), jnp.float32),
  )(u, p, q, wdanT, bdanC, wencT, bencC, wdecT, bdecC,
    ws1T, ws2T, wt1T, wt2T, bs, bt)


def kernel(SInterBatch, TInterBatch, SUserTable, TUserTable, SItemTable,
           TItemTable, W_dan, b_dan, W_srec, b_srec, W_trec, b_trec,
           W_enc, b_enc, W_dec, b_dec):
  idx = lambda a: a.astype(jnp.int32).reshape(IDX_ROWS, CHUNK)
  u, p, q = _sc_gather(
      SUserTable, TUserTable, SItemTable, TItemTable,
      idx(SInterBatch[0]), idx(TInterBatch[0]),
      idx(SInterBatch[1]), idx(SInterBatch[2]),
      idx(TInterBatch[1]), idx(TInterBatch[2]))

  outs = _tc_heads(
      u, p, q,
      W_dan.T, b_dan.reshape(-1, 1), W_enc.T, b_enc.reshape(-1, 1),
      W_dec.T, b_dec.reshape(-1, 1),
      W_srec[:32].T, W_srec[32:].T, W_trec[:32].T, W_trec[32:].T,
      b_srec.reshape(1, 1), b_trec.reshape(1, 1))
  return outs.reshape(8 * B, 1)


# speedup vs baseline: 2.1065x; 1.1791x over previous
import functools

import jax
import jax.numpy as jnp
from jax import lax
from jax.experimental import pallas as pl
from jax.experimental.pallas import tpu as pltpu
from jax.experimental.pallas import tpu_sc as plsc

B = 16384
D = 64
NW = 32
ROWS_PER_W = B // NW
CHUNK = 128
CHUNKS_PER_W = ROWS_PER_W // CHUNK
IDX_ROWS = B // CHUNK


def _sc_mesh_kernel(body, n_out):
  out2 = jax.ShapeDtypeStruct((B, 2 * D), jnp.float32)
  mesh = plsc.VectorSubcoreMesh(core_axis_name="c", subcore_axis_name="s")
  return functools.partial(
      pl.kernel,
      mesh=mesh,
      compiler_params=pltpu.CompilerParams(use_tc_tiling_on_sc=False),
      out_type=[out2] * n_out,
      scratch_types=[
          pltpu.VMEM((CHUNKS_PER_W, CHUNK), jnp.int32),
          pltpu.VMEM((ROWS_PER_W, D), jnp.float32),
          pltpu.SemaphoreType.DMA,
      ],
  )(body)


def _gather_jobs(jobs, idx_v, rows_v, sem):
  wid = lax.axis_index("s") * 2 + lax.axis_index("c")
  base = wid * CHUNKS_PER_W
  row0 = wid * ROWS_PER_W
  for tab, idx, out, col in jobs:
    pltpu.sync_copy(idx.at[pl.ds(base, CHUNKS_PER_W), :], idx_v)
    copies = [
        pltpu.async_copy(tab.at[idx_v.at[j]],
                         rows_v.at[pl.ds(j * CHUNK, CHUNK), :], sem)
        for j in range(CHUNKS_PER_W)
    ]
    for c in copies:
      c.wait()
    pltpu.sync_copy(rows_v, out.at[pl.ds(row0, ROWS_PER_W), pl.ds(col, D)])


def _sc_items_body(si_t, ti_t, i_sp, i_sn, i_tp, i_tn, o_s, o_t,
                   idx_v, rows_v, sem):
  _gather_jobs(((si_t, i_sp, o_s, 0), (si_t, i_sn, o_s, D),
                (ti_t, i_tp, o_t, 0), (ti_t, i_tn, o_t, D)),
               idx_v, rows_v, sem)


def _sc_users_body(su_t, tu_t, i_su, i_tu, o_u, idx_v, rows_v, sem):
  _gather_jobs(((su_t, i_su, o_u, 0), (tu_t, i_tu, o_u, D)),
               idx_v, rows_v, sem)


BLK = 2048


def _tc_body(u, p, q, wdanT, bdanC, wencT, bencC, wdecT, bdecC,
             w_user, w_item_s, w_item_t, bvec, out):
  dn = (((1,), (0,)), ((), ()))
  dt = (((1,), (1,)), ((), ()))
  dot = functools.partial(lax.dot_general,
                          preferred_element_type=jnp.float32)
  su = u[:, :D]
  tu = u[:, D:]
  sdiT = jnp.maximum(dot(wdanT[...], su, dt) + bdanC[...], 0.0)
  tdiT = jnp.maximum(dot(wdanT[...], tu, dt) + bdanC[...], 0.0)
  scdT = dot(wdecT[...],
             jnp.maximum(dot(wencT[...], sdiT, dn) + bencC[...], 0.0),
             dn) + bdecC[...]
  tcdT = dot(wdecT[...],
             jnp.maximum(dot(wencT[...], tdiT, dn) + bencC[...], 0.0),
             dn) + bdecC[...]
  ustack = jnp.concatenate([sdiT, scdT, tdiT, tcdT], axis=0)
  uh = dot(w_user[...], ustack, dn)
  sh = dot(w_item_s[...], p[...], dt)
  th = dot(w_item_t[...], q[...], dt)
  a_s, c_s, a_t, c_t = uh[0:1], uh[1:2], uh[2:3], uh[3:4]
  p_s, n_s = sh[0:1], sh[1:2]
  p_t, n_t = th[0:1], th[1:2]
  bs = bvec[0, 0]
  bt = bvec[0, 1]
  out[...] = jnp.concatenate(
      [a_s + p_s + bs, a_t + p_t + bt,
       a_s + n_s + bs, a_t + n_t + bt,
       c_s + p_s + bs, c_s + n_s + bs,
       c_t + p_t + bt, c_t + n_t + bt], axis=0)


def _tc_heads(u, p, q, wdanT, bdanC, wencT, bencC, wdecT, bdecC,
              w_user, w_item_s, w_item_t, bvec):
  row_spec = pl.BlockSpec((BLK, 2 * D), lambda i: (i, 0))
  full = lambda a: pl.BlockSpec(a.shape, lambda i: (0,) * a.ndim)
  return pl.pallas_call(
      _tc_body,
      grid=(B // BLK,),
      in_specs=[row_spec] * 3 + [full(w) for w in
                                 (wdanT, bdanC, wencT, bencC, wdecT, bdecC,
                                  w_user, w_item_s, w_item_t, bvec)],
      out_specs=pl.BlockSpec((8, BLK), lambda i: (0, i)),
      out_shape=jax.ShapeDtypeStruct((8, B), jnp.float32),
  )(u, p, q, wdanT, bdanC, wencT, bencC, wdecT, bdecC,
    w_user, w_item_s, w_item_t, bvec)


def kernel(SInterBatch, TInterBatch, SUserTable, TUserTable, SItemTable,
           TItemTable, W_dan, b_dan, W_srec, b_srec, W_trec, b_trec,
           W_enc, b_enc, W_dec, b_dec):
  idx = lambda a: a.astype(jnp.int32).reshape(IDX_ROWS, CHUNK)
  items_k = _sc_mesh_kernel(_sc_items_body, 2)
  users_k = _sc_mesh_kernel(_sc_users_body, 1)
  p, q = items_k(SItemTable, TItemTable,
                 idx(SInterBatch[1]), idx(SInterBatch[2]),
                 idx(TInterBatch[1]), idx(TInterBatch[2]))
  (u,) = users_k(SUserTable, TUserTable,
                 idx(SInterBatch[0]), idx(TInterBatch[0]))

  z32 = jnp.zeros((1, 32), jnp.float32)
  ws1T, wt1T = W_srec[:32].T, W_trec[:32].T
  w_user = jnp.concatenate([
      jnp.concatenate([ws1T, z32, z32, z32], axis=1),
      jnp.concatenate([z32, ws1T, z32, z32], axis=1),
      jnp.concatenate([z32, z32, wt1T, z32], axis=1),
      jnp.concatenate([z32, z32, z32, wt1T], axis=1)], axis=0)
  z64 = jnp.zeros((1, D), jnp.float32)
  ws2T, wt2T = W_srec[32:].T, W_trec[32:].T
  w_item_s = jnp.concatenate([
      jnp.concatenate([ws2T, z64], axis=1),
      jnp.concatenate([z64, ws2T], axis=1)], axis=0)
  w_item_t = jnp.concatenate([
      jnp.concatenate([wt2T, z64], axis=1),
      jnp.concatenate([z64, wt2T], axis=1)], axis=0)
  bvec = jnp.stack([b_srec[0], b_trec[0]]).reshape(1, 2)

  outs = _tc_heads(
      u, p, q,
      W_dan.T, b_dan.reshape(-1, 1), W_enc.T, b_enc.reshape(-1, 1),
      W_dec.T, b_dec.reshape(-1, 1),
      w_user, w_item_s, w_item_t, bvec)
  return outs.reshape(8 * B, 1)


# speedup vs baseline: 2.1287x; 1.0105x over previous
import functools

import jax
import jax.numpy as jnp
from jax import lax
from jax.experimental import pallas as pl
from jax.experimental.pallas import tpu as pltpu
from jax.experimental.pallas import tpu_sc as plsc

B = 16384
D = 64
NW = 32
ROWS_PER_W = B // NW
CHUNK = 128
CHUNKS_PER_W = ROWS_PER_W // CHUNK
IDX_ROWS = B // CHUNK


def _sc_mesh_kernel(body, n_out):
  out2 = jax.ShapeDtypeStruct((B, 2 * D), jnp.float32)
  mesh = plsc.VectorSubcoreMesh(core_axis_name="c", subcore_axis_name="s")
  return functools.partial(
      pl.kernel,
      mesh=mesh,
      compiler_params=pltpu.CompilerParams(use_tc_tiling_on_sc=False),
      out_type=[out2] * n_out,
      scratch_types=[
          pltpu.VMEM((CHUNKS_PER_W, CHUNK), jnp.int32),
          pltpu.VMEM((ROWS_PER_W, D), jnp.float32),
          pltpu.SemaphoreType.DMA,
      ],
  )(body)


def _gather_jobs(jobs, idx_v, rows_v, sem):
  wid = lax.axis_index("s") * 2 + lax.axis_index("c")
  base = wid * CHUNKS_PER_W
  row0 = wid * ROWS_PER_W
  for tab, idx, out, col in jobs:
    pltpu.sync_copy(idx.at[pl.ds(base, CHUNKS_PER_W), :], idx_v)
    copies = [
        pltpu.async_copy(tab.at[idx_v.at[j]],
                         rows_v.at[pl.ds(j * CHUNK, CHUNK), :], sem)
        for j in range(CHUNKS_PER_W)
    ]
    for c in copies:
      c.wait()
    pltpu.sync_copy(rows_v, out.at[pl.ds(row0, ROWS_PER_W), pl.ds(col, D)])


def _sc_items_body(si_t, ti_t, i_sp, i_sn, i_tp, i_tn, o_s, o_t,
                   idx_v, rows_v, sem):
  _gather_jobs(((si_t, i_sp, o_s, 0), (si_t, i_sn, o_s, D),
                (ti_t, i_tp, o_t, 0), (ti_t, i_tn, o_t, D)),
               idx_v, rows_v, sem)


def _sc_su_body(su_t, i_su, o_u, idx_v, rows_v, sem):
  _gather_jobs(((su_t, i_su, o_u, 0),), idx_v, rows_v, sem)


def _sc_tu_body(tu_t, i_tu, o_u, idx_v, rows_v, sem):
  _gather_jobs(((tu_t, i_tu, o_u, D),), idx_v, rows_v, sem)


BLK = 2048


def _tc_body(us, ut, p, q, wdanT, bdanC, wencT, bencC, wdecT, bdecC,
             w_user, w_item_s, w_item_t, bvec, out):
  dn = (((1,), (0,)), ((), ()))
  dt = (((1,), (1,)), ((), ()))
  dot = functools.partial(lax.dot_general,
                          preferred_element_type=jnp.float32)
  su = us[:, :D]
  tu = ut[:, D:]
  sdiT = jnp.maximum(dot(wdanT[...], su, dt) + bdanC[...], 0.0)
  tdiT = jnp.maximum(dot(wdanT[...], tu, dt) + bdanC[...], 0.0)
  scdT = dot(wdecT[...],
             jnp.maximum(dot(wencT[...], sdiT, dn) + bencC[...], 0.0),
             dn) + bdecC[...]
  tcdT = dot(wdecT[...],
             jnp.maximum(dot(wencT[...], tdiT, dn) + bencC[...], 0.0),
             dn) + bdecC[...]
  ustack = jnp.concatenate([sdiT, scdT, tdiT, tcdT], axis=0)
  uh = dot(w_user[...], ustack, dn)
  sh = dot(w_item_s[...], p[...], dt)
  th = dot(w_item_t[...], q[...], dt)
  a_s, c_s, a_t, c_t = uh[0:1], uh[1:2], uh[2:3], uh[3:4]
  p_s, n_s = sh[0:1], sh[1:2]
  p_t, n_t = th[0:1], th[1:2]
  bs = bvec[0, 0]
  bt = bvec[0, 1]
  out[...] = jnp.concatenate(
      [a_s + p_s + bs, a_t + p_t + bt,
       a_s + n_s + bs, a_t + n_t + bt,
       c_s + p_s + bs, c_s + n_s + bs,
       c_t + p_t + bt, c_t + n_t + bt], axis=0)


def _tc_heads(us, ut, p, q, wdanT, bdanC, wencT, bencC, wdecT, bdecC,
              w_user, w_item_s, w_item_t, bvec):
  row_spec = pl.BlockSpec((BLK, 2 * D), lambda i: (i, 0))
  full = lambda a: pl.BlockSpec(a.shape, lambda i: (0,) * a.ndim)
  return pl.pallas_call(
      _tc_body,
      grid=(B // BLK,),
      in_specs=[row_spec] * 4 + [full(w) for w in
                                 (wdanT, bdanC, wencT, bencC, wdecT, bdecC,
                                  w_user, w_item_s, w_item_t, bvec)],
      out_specs=pl.BlockSpec((8, BLK), lambda i: (0, i)),
      out_shape=jax.ShapeDtypeStruct((8, B), jnp.float32),
  )(us, ut, p, q, wdanT, bdanC, wencT, bencC, wdecT, bdecC,
    w_user, w_item_s, w_item_t, bvec)


def kernel(SInterBatch, TInterBatch, SUserTable, TUserTable, SItemTable,
           TItemTable, W_dan, b_dan, W_srec, b_srec, W_trec, b_trec,
           W_enc, b_enc, W_dec, b_dec):
  idx = lambda a: a.astype(jnp.int32).reshape(IDX_ROWS, CHUNK)
  items_k = _sc_mesh_kernel(_sc_items_body, 2)
  su_k = _sc_mesh_kernel(_sc_su_body, 1)
  tu_k = _sc_mesh_kernel(_sc_tu_body, 1)
  p, q = items_k(SItemTable, TItemTable,
                 idx(SInterBatch[1]), idx(SInterBatch[2]),
                 idx(TInterBatch[1]), idx(TInterBatch[2]))
  (us,) = su_k(SUserTable, idx(SInterBatch[0]))
  (ut,) = tu_k(TUserTable, idx(TInterBatch[0]))

  z32 = jnp.zeros((1, 32), jnp.float32)
  ws1T, wt1T = W_srec[:32].T, W_trec[:32].T
  w_user = jnp.concatenate([
      jnp.concatenate([ws1T, z32, z32, z32], axis=1),
      jnp.concatenate([z32, ws1T, z32, z32], axis=1),
      jnp.concatenate([z32, z32, wt1T, z32], axis=1),
      jnp.concatenate([z32, z32, z32, wt1T], axis=1)], axis=0)
  z64 = jnp.zeros((1, D), jnp.float32)
  ws2T, wt2T = W_srec[32:].T, W_trec[32:].T
  w_item_s = jnp.concatenate([
      jnp.concatenate([ws2T, z64], axis=1),
      jnp.concatenate([z64, ws2T], axis=1)], axis=0)
  w_item_t = jnp.concatenate([
      jnp.concatenate([wt2T, z64], axis=1),
      jnp.concatenate([z64, wt2T], axis=1)], axis=0)
  bvec = jnp.stack([b_srec[0], b_trec[0]]).reshape(1, 2)

  outs = _tc_heads(
      us, ut, p, q,
      W_dan.T, b_dan.reshape(-1, 1), W_enc.T, b_enc.reshape(-1, 1),
      W_dec.T, b_dec.reshape(-1, 1),
      w_user, w_item_s, w_item_t, bvec)
  return outs.reshape(8 * B, 1)


# speedup vs baseline: 3.2098x; 1.5079x over previous
import functools

import jax
import jax.numpy as jnp
from jax import lax
from jax.experimental import pallas as pl
from jax.experimental.pallas import tpu as pltpu
from jax.experimental.pallas import tpu_sc as plsc

B = 16384
D = 64
NW = 32
ROWS_PER_W = B // NW
CHUNK = 128
CHUNKS_PER_W = ROWS_PER_W // CHUNK
IDX_ROWS = B // CHUNK
V = 100000
DOT_W = 100096
NSTEP = 8


def _tc_dots_body(siT, tiT, ws2C, wt2C, out, accS, accT):
  pid = pl.program_id(0)

  @pl.when(pid == 0)
  def _init():
    accS[...] = jnp.zeros_like(accS)
    accT[...] = jnp.zeros_like(accT)

  accS[...] += siT[...] * ws2C[pl.ds(pid * 8, 8), :]
  accT[...] += tiT[...] * wt2C[pl.ds(pid * 8, 8), :]

  @pl.when(pid == NSTEP - 1)
  def _fin():
    out[0:1, pl.ds(0, V)] = jnp.sum(accS[...], axis=0, keepdims=True)
    out[1:2, pl.ds(0, V)] = jnp.sum(accT[...], axis=0, keepdims=True)


def _tc_dots(siT, tiT, ws2C, wt2C):
  full = lambda a: pl.BlockSpec(a.shape, lambda i: (0,) * a.ndim)
  return pl.pallas_call(
      _tc_dots_body,
      grid=(NSTEP,),
      in_specs=[pl.BlockSpec((8, V), lambda i: (i, 0))] * 2
      + [full(ws2C), full(wt2C)],
      out_specs=pl.BlockSpec((8, DOT_W), lambda i: (0, 0)),
      out_shape=jax.ShapeDtypeStruct((8, DOT_W), jnp.float32),
      scratch_shapes=[pltpu.VMEM((8, V), jnp.float32)] * 2,
  )(siT, tiT, ws2C, wt2C)


def _sc_mesh_kernel(body, out_types, scratch_types):
  mesh = plsc.VectorSubcoreMesh(core_axis_name="c", subcore_axis_name="s")
  return functools.partial(
      pl.kernel,
      mesh=mesh,
      compiler_params=pltpu.CompilerParams(use_tc_tiling_on_sc=False),
      out_type=out_types,
      scratch_types=scratch_types + [pltpu.SemaphoreType.DMA],
  )(body)


def _sc_dots_body(dots, i_sp, i_sn, i_tp, i_tn, o_d, idx_v, val_v, sem):
  wid = lax.axis_index("s") * 2 + lax.axis_index("c")
  base = wid * CHUNKS_PER_W
  row0 = wid * ROWS_PER_W
  for r, idx in enumerate((i_sp, i_sn, i_tp, i_tn)):
    pltpu.sync_copy(idx.at[pl.ds(base, CHUNKS_PER_W), :], idx_v)
    copies = [
        pltpu.async_copy(dots.at[idx_v.at[j]],
                         val_v.at[pl.ds(j * CHUNK, CHUNK)], sem)
        for j in range(CHUNKS_PER_W)
    ]
    for c in copies:
      c.wait()
    pltpu.sync_copy(val_v, o_d.at[r, pl.ds(row0, ROWS_PER_W)])


def _sc_dots(dots_flat, i_sp, i_sn, i_tp, i_tn):
  k = _sc_mesh_kernel(
      _sc_dots_body,
      [jax.ShapeDtypeStruct((8, B), jnp.float32)],
      [pltpu.VMEM((CHUNKS_PER_W, CHUNK), jnp.int32),
       pltpu.VMEM((ROWS_PER_W,), jnp.float32)])
  return k(dots_flat, i_sp, i_sn, i_tp, i_tn)


def _user_gather_body(tab, idx, out, col, idx_v, rows_v, sem):
  wid = lax.axis_index("s") * 2 + lax.axis_index("c")
  base = wid * CHUNKS_PER_W
  row0 = wid * ROWS_PER_W
  pltpu.sync_copy(idx.at[pl.ds(base, CHUNKS_PER_W), :], idx_v)
  copies = [
      pltpu.async_copy(tab.at[idx_v.at[j]],
                       rows_v.at[pl.ds(j * CHUNK, CHUNK), :], sem)
      for j in range(CHUNKS_PER_W)
  ]
  for c in copies:
    c.wait()
  pltpu.sync_copy(rows_v, out.at[pl.ds(row0, ROWS_PER_W), pl.ds(col, D)])


def _sc_su_body(su_t, i_su, o_u, idx_v, rows_v, sem):
  _user_gather_body(su_t, i_su, o_u, 0, idx_v, rows_v, sem)


def _sc_tu_body(tu_t, i_tu, o_u, idx_v, rows_v, sem):
  _user_gather_body(tu_t, i_tu, o_u, D, idx_v, rows_v, sem)


def _sc_user(body, tab, idx):
  k = _sc_mesh_kernel(
      body,
      [jax.ShapeDtypeStruct((B, 2 * D), jnp.float32)],
      [pltpu.VMEM((CHUNKS_PER_W, CHUNK), jnp.int32),
       pltpu.VMEM((ROWS_PER_W, D), jnp.float32)])
  return k(tab, idx)


BLK = 2048


def _tc_body(us, ut, dots, wdanT, bdanC, wencT, bencC, wdecT, bdecC,
             w_user, bvec, out):
  dn = (((1,), (0,)), ((), ()))
  dt = (((1,), (1,)), ((), ()))
  dot = functools.partial(lax.dot_general,
                          preferred_element_type=jnp.float32)
  su = us[:, :D]
  tu = ut[:, D:]
  sdiT = jnp.maximum(dot(wdanT[...], su, dt) + bdanC[...], 0.0)
  tdiT = jnp.maximum(dot(wdanT[...], tu, dt) + bdanC[...], 0.0)
  scdT = dot(wdecT[...],
             jnp.maximum(dot(wencT[...], sdiT, dn) + bencC[...], 0.0),
             dn) + bdecC[...]
  tcdT = dot(wdecT[...],
             jnp.maximum(dot(wencT[...], tdiT, dn) + bencC[...], 0.0),
             dn) + bdecC[...]
  ustack = jnp.concatenate([sdiT, scdT, tdiT, tcdT], axis=0)
  uh = dot(w_user[...], ustack, dn)
  a_s, c_s, a_t, c_t = uh[0:1], uh[1:2], uh[2:3], uh[3:4]
  d = dots[...]
  p_s, n_s, p_t, n_t = d[0:1], d[1:2], d[2:3], d[3:4]
  bs = bvec[0, 0]
  bt = bvec[0, 1]
  out[...] = jnp.concatenate(
      [a_s + p_s + bs, a_t + p_t + bt,
       a_s + n_s + bs, a_t + n_t + bt,
       c_s + p_s + bs, c_s + n_s + bs,
       c_t + p_t + bt, c_t + n_t + bt], axis=0)


def _tc_heads(us, ut, dots, wdanT, bdanC, wencT, bencC, wdecT, bdecC,
              w_user, bvec):
  row_spec = pl.BlockSpec((BLK, 2 * D), lambda i: (i, 0))
  full = lambda a: pl.BlockSpec(a.shape, lambda i: (0,) * a.ndim)
  return pl.pallas_call(
      _tc_body,
      grid=(B // BLK,),
      in_specs=[row_spec] * 2 + [pl.BlockSpec((8, BLK), lambda i: (0, i))]
      + [full(w) for w in (wdanT, bdanC, wencT, bencC, wdecT, bdecC,
                           w_user, bvec)],
      out_specs=pl.BlockSpec((8, BLK), lambda i: (0, i)),
      out_shape=jax.ShapeDtypeStruct((8, B), jnp.float32),
  )(us, ut, dots, wdanT, bdanC, wencT, bencC, wdecT, bdecC, w_user, bvec)


def kernel(SInterBatch, TInterBatch, SUserTable, TUserTable, SItemTable,
           TItemTable, W_dan, b_dan, W_srec, b_srec, W_trec, b_trec,
           W_enc, b_enc, W_dec, b_dec):
  idx = lambda a: a.astype(jnp.int32).reshape(IDX_ROWS, CHUNK)

  def dotpos(a, row):
    return (a.astype(jnp.int32) + row * DOT_W).reshape(IDX_ROWS, CHUNK)

  dots8 = _tc_dots(SItemTable.T, TItemTable.T, W_srec[32:], W_trec[32:])
  dots_flat = dots8.reshape(8 * DOT_W)
  dots = _sc_dots(dots_flat,
                  dotpos(SInterBatch[1], 0), dotpos(SInterBatch[2], 0),
                  dotpos(TInterBatch[1], 1), dotpos(TInterBatch[2], 1))
  (us,) = _sc_user(_sc_su_body, SUserTable, idx(SInterBatch[0]))
  (ut,) = _sc_user(_sc_tu_body, TUserTable, idx(TInterBatch[0]))

  z32 = jnp.zeros((1, 32), jnp.float32)
  ws1T, wt1T = W_srec[:32].T, W_trec[:32].T
  w_user = jnp.concatenate([
      jnp.concatenate([ws1T, z32, z32, z32], axis=1),
      jnp.concatenate([z32, ws1T, z32, z32], axis=1),
      jnp.concatenate([z32, z32, wt1T, z32], axis=1),
      jnp.concatenate([z32, z32, z32, wt1T], axis=1)], axis=0)
  bvec = jnp.stack([b_srec[0], b_trec[0]]).reshape(1, 2)

  outs = _tc_heads(
      us, ut, dots[0],
      W_dan.T, b_dan.reshape(-1, 1), W_enc.T, b_enc.reshape(-1, 1),
      W_dec.T, b_dec.reshape(-1, 1), w_user, bvec)
  return outs.reshape(8 * B, 1)
